# Initial kernel scaffold; baseline (speedup 1.0000x reference)
import jax
import jax.numpy as jnp
from jax.experimental import pallas as pl


def kernel(x, table):
    raise NotImplementedError("write your pallas kernel here")


# speedup vs baseline: 4.1286x; 4.1286x over previous
import functools

import jax
import jax.numpy as jnp
from jax import lax
from jax.experimental import pallas as pl
from jax.experimental.pallas import tpu as pltpu
from jax.experimental.pallas import tpu_sc as plsc

N_EMB = 100000
EMB_DIM = 128
N_HASHES = 4
B = 1024
L = 200

N_TOK = B * L
NC, NS, LANES = 2, 16, 16
NW = NC * NS
TOK_PER_W = N_TOK // NW
T = 64
G = (T * N_HASHES) // 128
CHUNKS = TOK_PER_W // T
IDX_ROW = 128 // N_HASHES


@functools.partial(
    pl.kernel,
    out_type=jax.ShapeDtypeStruct((N_TOK, EMB_DIM), jnp.float32),
    mesh=plsc.VectorSubcoreMesh(
        core_axis_name="c", subcore_axis_name="s", num_cores=NC,
        num_subcores=NS),
    scratch_types=[
        pltpu.VMEM((TOK_PER_W * N_HASHES // 128, 128), jnp.int32),
        pltpu.VMEM((T * N_HASHES, EMB_DIM), jnp.float32),
        pltpu.VMEM((T, EMB_DIM), jnp.float32),
        pltpu.SemaphoreType.DMA,
    ],
)
def _bloom_sum(x_hbm, table_hbm, out_hbm, idx_v, rows_v, out_v, sem):
    wid = lax.axis_index("s") * NC + lax.axis_index("c")
    tok0 = wid * TOK_PER_W
    idx_rows = TOK_PER_W * N_HASHES // 128
    pltpu.sync_copy(x_hbm.at[pl.ds(wid * idx_rows, idx_rows)], idx_v)

    def chunk_body(c, carry):
        tok_base = tok0 + c * T
        cps = [
            pltpu.async_copy(
                table_hbm.at[idx_v.at[c * G + g]],
                rows_v.at[pl.ds(g * 128, 128)], sem)
            for g in range(G)
        ]
        for cp in cps:
            cp.wait()

        def tok_body(t, tc):
            r = t * N_HASHES
            for d in range(EMB_DIM // LANES):
                s = pl.ds(d * LANES, LANES)
                acc = rows_v[r, s] + rows_v[r + 1, s]
                acc = acc + rows_v[r + 2, s]
                acc = acc + rows_v[r + 3, s]
                out_v[t, s] = acc
            return tc

        lax.fori_loop(0, T, tok_body, 0, unroll=False)
        pltpu.sync_copy(out_v, out_hbm.at[pl.ds(tok_base, T)])
        return carry

    lax.fori_loop(0, CHUNKS, chunk_body, 0, unroll=False)


def kernel(x, table):
    xf = x.reshape(N_TOK * N_HASHES // 128, 128)
    out = _bloom_sum(xf, table)
    return out.reshape(B, L, EMB_DIM)


# speedup vs baseline: 5.9439x; 1.4397x over previous
import functools

import jax
import jax.numpy as jnp
from jax import lax
from jax.experimental import pallas as pl
from jax.experimental.pallas import tpu as pltpu
from jax.experimental.pallas import tpu_sc as plsc

N_EMB = 100000
EMB_DIM = 128
N_HASHES = 4
B = 1024
L = 200

N_TOK = B * L
NC, NS, LANES = 2, 16, 16
NW = NC * NS
TOK_PER_W = N_TOK // NW
T = 64
G = (T * N_HASHES) // 128
CHUNKS = TOK_PER_W // T
IDX_ROW = 128 // N_HASHES


@functools.partial(
    pl.kernel,
    out_type=jax.ShapeDtypeStruct((N_TOK, EMB_DIM), jnp.float32),
    mesh=plsc.VectorSubcoreMesh(
        core_axis_name="c", subcore_axis_name="s", num_cores=NC,
        num_subcores=NS),
    scratch_types=[
        pltpu.VMEM((TOK_PER_W * N_HASHES // 128, 128), jnp.int32),
        pltpu.VMEM((2, T * N_HASHES, EMB_DIM), jnp.float32),
        pltpu.VMEM((2, T, EMB_DIM), jnp.float32),
        pltpu.SemaphoreType.DMA,
        pltpu.SemaphoreType.DMA,
        pltpu.SemaphoreType.DMA,
        pltpu.SemaphoreType.DMA,
    ],
)
def _bloom_sum(x_hbm, table_hbm, out_hbm, idx_v, rows_v, out_v,
               gsem0, gsem1, ssem0, ssem1):
    wid = lax.axis_index("s") * NC + lax.axis_index("c")
    tok0 = wid * TOK_PER_W
    idx_rows = TOK_PER_W * N_HASHES // 128
    pltpu.sync_copy(x_hbm.at[pl.ds(wid * idx_rows, idx_rows)], idx_v)
    gsems = [gsem0, gsem1]
    ssems = [ssem0, ssem1]

    def issue_gather(c, b):
        for g in range(G):
            pltpu.async_copy(
                table_hbm.at[idx_v.at[c * G + g]],
                rows_v.at[b].at[pl.ds(g * 128, 128)], gsems[b])

    def wait_gather(b):
        for g in range(G):
            pltpu.make_async_copy(
                table_hbm.at[idx_v.at[g]],
                rows_v.at[b].at[pl.ds(g * 128, 128)], gsems[b]).wait()

    def wait_store(b):
        pltpu.make_async_copy(
            out_v.at[b], out_hbm.at[pl.ds(tok0, T)], ssems[b]).wait()

    issue_gather(0, 0)

    def super_body(c2, carry):
        for b in range(2):
            c = c2 + b

            @pl.when(c + 1 < CHUNKS)
            def _():
                issue_gather(c + 1, 1 - b)

            wait_gather(b)

            @pl.when(c >= 2)
            def _():
                wait_store(b)

            def tok_body(t, tc):
                r = t * N_HASHES
                for d in range(EMB_DIM // LANES):
                    s = pl.ds(d * LANES, LANES)
                    acc = rows_v[b, r, s] + rows_v[b, r + 1, s]
                    acc = acc + rows_v[b, r + 2, s]
                    acc = acc + rows_v[b, r + 3, s]
                    out_v[b, t, s] = acc
                return tc

            lax.fori_loop(0, T, tok_body, 0, unroll=False)
            pltpu.async_copy(
                out_v.at[b], out_hbm.at[pl.ds(tok0 + c * T, T)], ssems[b])
        return carry

    lax.fori_loop(0, CHUNKS // 2, lambda i, cr: super_body(i * 2, cr), 0,
                  unroll=False)
    wait_store(0)
    wait_store(1)


def kernel(x, table):
    xf = x.reshape(N_TOK * N_HASHES // 128, 128)
    out = _bloom_sum(xf, table)
    return out.reshape(B, L, EMB_DIM)


# speedup vs baseline: 8.4500x; 1.4216x over previous
import functools

import jax
import jax.numpy as jnp
from jax import lax
from jax.experimental import pallas as pl
from jax.experimental.pallas import tpu as pltpu
from jax.experimental.pallas import tpu_sc as plsc

N_EMB = 100000
EMB_DIM = 128
N_HASHES = 4
B = 1024
L = 200

N_TOK = B * L
NC, NS, LANES = 2, 16, 16
NW = NC * NS
TOK_PER_W = N_TOK // NW
T = 64
G = (T * N_HASHES) // 128
CHUNKS = TOK_PER_W // T
IDX_ROW = 128 // N_HASHES


@functools.partial(
    pl.kernel,
    out_type=jax.ShapeDtypeStruct((N_TOK, EMB_DIM), jnp.float32),
    mesh=plsc.VectorSubcoreMesh(
        core_axis_name="c", subcore_axis_name="s", num_cores=NC,
        num_subcores=NS),
    scratch_types=[
        pltpu.VMEM((TOK_PER_W * N_HASHES // 128, 128), jnp.int32),
        pltpu.VMEM((2, T * N_HASHES, EMB_DIM), jnp.float32),
        pltpu.VMEM((2, T, EMB_DIM), jnp.float32),
        pltpu.SemaphoreType.DMA,
        pltpu.SemaphoreType.DMA,
        pltpu.SemaphoreType.DMA,
        pltpu.SemaphoreType.DMA,
    ],
)
def _bloom_sum(x3_hbm, table_hbm, out_hbm, idx_v, rows_v, out_v,
               gsem0, gsem1, ssem0, ssem1):
    x_hbm = x3_hbm.reshape(N_TOK * N_HASHES // 128, 128)
    wid = lax.axis_index("s") * NC + lax.axis_index("c")
    tok0 = wid * TOK_PER_W
    idx_rows = TOK_PER_W * N_HASHES // 128
    pltpu.sync_copy(x_hbm.at[pl.ds(wid * idx_rows, idx_rows)], idx_v)
    idx_r = idx_v
    gsems = [gsem0, gsem1]
    ssems = [ssem0, ssem1]

    def issue_gather(c, b):
        for g in range(G):
            pltpu.async_copy(
                table_hbm.at[idx_r.at[c * G + g]],
                rows_v.at[b].at[pl.ds(g * 128, 128)], gsems[b])

    def wait_gather(b):
        for g in range(G):
            pltpu.make_async_copy(
                table_hbm.at[idx_r.at[g]],
                rows_v.at[b].at[pl.ds(g * 128, 128)], gsems[b]).wait()

    def wait_store(b):
        pltpu.make_async_copy(
            out_v.at[b], out_hbm.at[pl.ds(tok0, T)], ssems[b]).wait()

    issue_gather(0, 0)

    def super_body(c2, carry):
        for b in range(2):
            c = c2 + b

            @pl.when(c + 1 < CHUNKS)
            def _():
                issue_gather(c + 1, 1 - b)

            wait_gather(b)

            @pl.when(c >= 2)
            def _():
                wait_store(b)

            def tok_body(t, tc):
                r = t * N_HASHES
                ngroups = EMB_DIM // LANES

                def loads(d):
                    return [rows_v[b, r + h, pl.ds(d * LANES, LANES)]
                            for h in range(N_HASHES)]

                cur = loads(0)
                for d in range(ngroups):
                    nxt = loads(d + 1) if d + 1 < ngroups else None
                    v0, v1, v2, v3 = cur
                    out_v[b, t, pl.ds(d * LANES, LANES)] = (
                        (v0 + v1) + (v2 + v3))
                    cur = nxt
                return tc

            lax.fori_loop(0, T, tok_body, 0, unroll=False)
            pltpu.async_copy(
                out_v.at[b], out_hbm.at[pl.ds(tok0 + c * T, T)], ssems[b])
        return carry

    lax.fori_loop(0, CHUNKS // 2, lambda i, cr: super_body(i * 2, cr), 0,
                  unroll=False)
    wait_store(0)
    wait_store(1)


def kernel(x, table):
    xf = x.reshape(N_TOK * N_HASHES // 128, 128)
    out = _bloom_sum(xf, table)
    return out.reshape(B, L, EMB_DIM)


# speedup vs baseline: 8.5081x; 1.0069x over previous
import functools

import jax
import jax.numpy as jnp
from jax import lax
from jax.experimental import pallas as pl
from jax.experimental.pallas import tpu as pltpu
from jax.experimental.pallas import tpu_sc as plsc

N_EMB = 100000
EMB_DIM = 128
N_HASHES = 4
B = 1024
L = 200

N_TOK = B * L
NC, NS, LANES = 2, 16, 16
NW = NC * NS
TOK_PER_W = N_TOK // NW
T = 64
G = (T * N_HASHES) // 128
CHUNKS = TOK_PER_W // T
IDX_ROW = 128 // N_HASHES


@functools.partial(
    pl.kernel,
    out_type=jax.ShapeDtypeStruct((N_TOK, EMB_DIM), jnp.float32),
    mesh=plsc.VectorSubcoreMesh(
        core_axis_name="c", subcore_axis_name="s", num_cores=NC,
        num_subcores=NS),
    scratch_types=[
        pltpu.VMEM((TOK_PER_W * N_HASHES // 128, 128), jnp.int32),
        pltpu.VMEM((2, T * N_HASHES, EMB_DIM), jnp.float32),
        pltpu.VMEM((2, T, EMB_DIM), jnp.float32),
        pltpu.SemaphoreType.DMA,
        pltpu.SemaphoreType.DMA,
        pltpu.SemaphoreType.DMA,
        pltpu.SemaphoreType.DMA,
    ],
)
def _bloom_sum(x3_hbm, table_hbm, out_hbm, idx_v, rows_v, out_v,
               gsem0, gsem1, ssem0, ssem1):
    x_hbm = x3_hbm.reshape(N_TOK * N_HASHES // 128, 128)
    wid = lax.axis_index("s") * NC + lax.axis_index("c")
    tok0 = wid * TOK_PER_W
    idx_rows = TOK_PER_W * N_HASHES // 128
    pltpu.sync_copy(x_hbm.at[pl.ds(wid * idx_rows, idx_rows)], idx_v)
    idx_r = idx_v
    gsems = [gsem0, gsem1]
    ssems = [ssem0, ssem1]

    def issue_gather(c, b):
        for g in range(G):
            pltpu.async_copy(
                table_hbm.at[idx_r.at[c * G + g]],
                rows_v.at[b].at[pl.ds(g * 128, 128)], gsems[b])

    def wait_gather(b):
        for g in range(G):
            pltpu.make_async_copy(
                table_hbm.at[idx_r.at[g]],
                rows_v.at[b].at[pl.ds(g * 128, 128)], gsems[b]).wait()

    def wait_store(b):
        pltpu.make_async_copy(
            out_v.at[b], out_hbm.at[pl.ds(tok0, T)], ssems[b]).wait()

    issue_gather(0, 0)

    def super_body(c2, carry):
        for b in range(2):
            c = c2 + b

            @pl.when(c + 1 < CHUNKS)
            def _():
                issue_gather(c + 1, 1 - b)

            wait_gather(b)

            @pl.when(c >= 2)
            def _():
                wait_store(b)

            def tok_body(t, tc):
                r = t * N_HASHES
                ngroups = EMB_DIM // LANES

                def loads(d):
                    return [rows_v[b, r + h, pl.ds(d * LANES, LANES)]
                            for h in range(N_HASHES)]

                cur = loads(0)
                for d in range(ngroups):
                    nxt = loads(d + 1) if d + 1 < ngroups else None
                    v0, v1, v2, v3 = cur
                    out_v[b, t, pl.ds(d * LANES, LANES)] = (
                        (v0 + v1) + (v2 + v3))
                    cur = nxt
                return tc

            lax.fori_loop(0, T, tok_body, 0, unroll=2)
            pltpu.async_copy(
                out_v.at[b], out_hbm.at[pl.ds(tok0 + c * T, T)], ssems[b])
        return carry

    lax.fori_loop(0, CHUNKS // 2, lambda i, cr: super_body(i * 2, cr), 0,
                  unroll=False)
    wait_store(0)
    wait_store(1)


def kernel(x, table):
    xf = x.reshape(N_TOK * N_HASHES // 128, 128)
    out = _bloom_sum(xf, table)
    return out.reshape(B, L, EMB_DIM)


# speedup vs baseline: 9.1027x; 1.0699x over previous
import functools

import jax
import jax.numpy as jnp
from jax import lax
from jax.experimental import pallas as pl
from jax.experimental.pallas import tpu as pltpu
from jax.experimental.pallas import tpu_sc as plsc

N_EMB = 100000
EMB_DIM = 128
N_HASHES = 4
B = 1024
L = 200

N_TOK = B * L
NC, NS, LANES = 2, 16, 16
NW = NC * NS
TOK_PER_W = N_TOK // NW
T = 64
G = (T * N_HASHES) // 128
CHUNKS = TOK_PER_W // T
IDX_ROW = 128 // N_HASHES


@functools.partial(
    pl.kernel,
    out_type=jax.ShapeDtypeStruct((N_TOK, EMB_DIM), jnp.float32),
    mesh=plsc.VectorSubcoreMesh(
        core_axis_name="c", subcore_axis_name="s", num_cores=NC,
        num_subcores=NS),
    scratch_types=[
        pltpu.VMEM((TOK_PER_W * N_HASHES // 128, 128), jnp.int32),
        pltpu.VMEM((2, T * N_HASHES, EMB_DIM), jnp.float32),
        pltpu.VMEM((2, T, EMB_DIM), jnp.float32),
        pltpu.SemaphoreType.DMA,
        pltpu.SemaphoreType.DMA,
        pltpu.SemaphoreType.DMA,
        pltpu.SemaphoreType.DMA,
    ],
)
def _bloom_sum(x3_hbm, table_hbm, out_hbm, idx_v, rows_v, out_v,
               gsem0, gsem1, ssem0, ssem1):
    x_hbm = x3_hbm.reshape(N_TOK * N_HASHES // 128, 128)
    wid = lax.axis_index("s") * NC + lax.axis_index("c")
    tok0 = wid * TOK_PER_W
    idx_rows = TOK_PER_W * N_HASHES // 128
    pltpu.sync_copy(x_hbm.at[pl.ds(wid * idx_rows, idx_rows)], idx_v)
    idx_r = idx_v
    gsems = [gsem0, gsem1]
    ssems = [ssem0, ssem1]

    def issue_gather(c, b):
        for g in range(G):
            pltpu.async_copy(
                table_hbm.at[idx_r.at[c * G + g]],
                rows_v.at[b].at[pl.ds(g * 128, 128)], gsems[b])

    def wait_gather(b):
        for g in range(G):
            pltpu.make_async_copy(
                table_hbm.at[idx_r.at[g]],
                rows_v.at[b].at[pl.ds(g * 128, 128)], gsems[b]).wait()

    def wait_store(b):
        pltpu.make_async_copy(
            out_v.at[b], out_hbm.at[pl.ds(tok0, T)], ssems[b]).wait()

    issue_gather(0, 0)

    def super_body(c2, carry):
        for b in range(2):
            c = c2 + b

            @pl.when(c + 1 < CHUNKS)
            def _():
                issue_gather(c + 1, 1 - b)

            wait_gather(b)

            @pl.when(c >= 2)
            def _():
                wait_store(b)

            def tok_body(t, tc):
                r = t * N_HASHES
                ngroups = EMB_DIM // LANES

                def loads(d):
                    return [rows_v[b, r + h, pl.ds(d * LANES, LANES)]
                            for h in range(N_HASHES)]

                cur = loads(0)
                nxt = loads(1)
                for d in range(ngroups):
                    nxt2 = loads(d + 2) if d + 2 < ngroups else None
                    v0, v1, v2, v3 = cur
                    out_v[b, t, pl.ds(d * LANES, LANES)] = (
                        (v0 + v1) + (v2 + v3))
                    cur, nxt = nxt, nxt2
                return tc

            lax.fori_loop(0, T, tok_body, 0, unroll=2)
            pltpu.async_copy(
                out_v.at[b], out_hbm.at[pl.ds(tok0 + c * T, T)], ssems[b])
        return carry

    lax.fori_loop(0, CHUNKS // 2, lambda i, cr: super_body(i * 2, cr), 0,
                  unroll=False)
    wait_store(0)
    wait_store(1)


def kernel(x, table):
    xf = x.reshape(N_TOK * N_HASHES // 128, 128)
    out = _bloom_sum(xf, table)
    return out.reshape(B, L, EMB_DIM)


# speedup vs baseline: 9.1621x; 1.0065x over previous
import functools

import jax
import jax.numpy as jnp
from jax import lax
from jax.experimental import pallas as pl
from jax.experimental.pallas import tpu as pltpu
from jax.experimental.pallas import tpu_sc as plsc

N_EMB = 100000
EMB_DIM = 128
N_HASHES = 4
B = 1024
L = 200

N_TOK = B * L
NC, NS, LANES = 2, 16, 16
NW = NC * NS
TOK_PER_W = N_TOK // NW
T = 64
G = (T * N_HASHES) // 128
CHUNKS = TOK_PER_W // T
IDX_ROW = 128 // N_HASHES


@functools.partial(
    pl.kernel,
    out_type=jax.ShapeDtypeStruct((N_TOK, EMB_DIM), jnp.float32),
    mesh=plsc.VectorSubcoreMesh(
        core_axis_name="c", subcore_axis_name="s", num_cores=NC,
        num_subcores=NS),
    scratch_types=[
        pltpu.VMEM((TOK_PER_W * N_HASHES // 128, 128), jnp.int32),
        pltpu.VMEM((2, T * N_HASHES, EMB_DIM), jnp.float32),
        pltpu.VMEM((2, T, EMB_DIM), jnp.float32),
        pltpu.SemaphoreType.DMA,
        pltpu.SemaphoreType.DMA,
        pltpu.SemaphoreType.DMA,
        pltpu.SemaphoreType.DMA,
    ],
)
def _bloom_sum(x3_hbm, table_hbm, out_hbm, idx_v, rows_v, out_v,
               gsem0, gsem1, ssem0, ssem1):
    x_hbm = x3_hbm.reshape(N_TOK * N_HASHES // 128, 128)
    wid = lax.axis_index("s") * NC + lax.axis_index("c")
    tok0 = wid * TOK_PER_W
    idx_rows = TOK_PER_W * N_HASHES // 128
    pltpu.sync_copy(x_hbm.at[pl.ds(wid * idx_rows, idx_rows)], idx_v)
    idx_r = idx_v
    gsems = [gsem0, gsem1]
    ssems = [ssem0, ssem1]

    def issue_gather(c, b):
        for g in range(G):
            pltpu.async_copy(
                table_hbm.at[idx_r.at[c * G + g]],
                rows_v.at[b].at[pl.ds(g * 128, 128)], gsems[b])

    def wait_gather(b):
        for g in range(G):
            pltpu.make_async_copy(
                table_hbm.at[idx_r.at[g]],
                rows_v.at[b].at[pl.ds(g * 128, 128)], gsems[b]).wait()

    def wait_store(b):
        pltpu.make_async_copy(
            out_v.at[b], out_hbm.at[pl.ds(tok0, T)], ssems[b]).wait()

    issue_gather(0, 0)

    def super_body(c2, carry):
        for b in range(2):
            c = c2 + b

            @pl.when(c + 1 < CHUNKS)
            def _():
                issue_gather(c + 1, 1 - b)

            wait_gather(b)

            @pl.when(c >= 2)
            def _():
                wait_store(b)

            def tok_body(t, tc):
                r = t * N_HASHES
                ngroups = EMB_DIM // LANES

                def loads(d):
                    return [rows_v[b, r + h, pl.ds(d * LANES, LANES)]
                            for h in range(N_HASHES)]

                pipe = [loads(0), loads(1), loads(2)]
                for d in range(ngroups):
                    if d + 3 < ngroups:
                        pipe.append(loads(d + 3))
                    v0, v1, v2, v3 = pipe.pop(0)
                    out_v[b, t, pl.ds(d * LANES, LANES)] = (
                        (v0 + v1) + (v2 + v3))
                return tc

            lax.fori_loop(0, T, tok_body, 0, unroll=2)
            pltpu.async_copy(
                out_v.at[b], out_hbm.at[pl.ds(tok0 + c * T, T)], ssems[b])
        return carry

    lax.fori_loop(0, CHUNKS // 2, lambda i, cr: super_body(i * 2, cr), 0,
                  unroll=False)
    wait_store(0)
    wait_store(1)


def kernel(x, table):
    xf = x.reshape(N_TOK * N_HASHES // 128, 128)
    out = _bloom_sum(xf, table)
    return out.reshape(B, L, EMB_DIM)


# speedup vs baseline: 15.2949x; 1.6694x over previous
import functools

import jax
import jax.numpy as jnp
from jax import lax
from jax.experimental import pallas as pl
from jax.experimental.pallas import tpu as pltpu
from jax.experimental.pallas import tpu_sc as plsc

N_EMB = 100000
EMB_DIM = 128
N_HASHES = 4
B = 1024
L = 200

N_TOK = B * L
NC, NS, LANES = 2, 16, 16
NW = NC * NS
BB = B // 128
NGRP = L * BB
GRP_PER_W = NGRP // NW
T = 64
ZROWS_PER_W = GRP_PER_W * N_HASHES


@functools.partial(
    pl.kernel,
    out_type=jax.ShapeDtypeStruct((N_TOK, EMB_DIM), jnp.float32),
    mesh=plsc.VectorSubcoreMesh(
        core_axis_name="c", subcore_axis_name="s", num_cores=NC,
        num_subcores=NS),
    scratch_types=[
        pltpu.VMEM((ZROWS_PER_W, 128), jnp.int32),
        pltpu.VMEM((2, N_HASHES * T, EMB_DIM), jnp.float32),
        pltpu.VMEM((2, T, EMB_DIM), jnp.float32),
        pltpu.VMEM((2, T), jnp.int32),
        pltpu.SemaphoreType.DMA,
        pltpu.SemaphoreType.DMA,
        pltpu.SemaphoreType.DMA,
        pltpu.SemaphoreType.DMA,
    ],
)
def _bloom_sum(z_hbm, table_hbm, out_hbm, idx_v, rows_v, out_v, offs_v,
               gsem0, gsem1, ssem0, ssem1):
    wid = lax.axis_index("s") * NC + lax.axis_index("c")
    pltpu.sync_copy(z_hbm.at[pl.ds(wid * ZROWS_PER_W, ZROWS_PER_W)], idx_v)
    g0 = wid * GRP_PER_W
    gsems = [gsem0, gsem1]
    ssems = [ssem0, ssem1]

    def issue_gather(gl, half, b):
        for h in range(N_HASHES):
            pltpu.async_copy(
                table_hbm.at[idx_v.at[gl * N_HASHES + h, pl.ds(half * T, T)]],
                rows_v.at[b].at[pl.ds(h * T, T)], gsems[b])

    def wait_gather(b):
        for h in range(N_HASHES):
            pltpu.make_async_copy(
                table_hbm.at[idx_v.at[h, pl.ds(0, T)]],
                rows_v.at[b].at[pl.ds(h * T, T)], gsems[b]).wait()

    def wait_store(b):
        pltpu.make_async_copy(
            out_v.at[b], out_hbm.at[offs_v.at[b]], ssems[b]).wait()

    issue_gather(0, 0, 0)

    def grp_body(gl, carry):
        g = g0 + gl
        l = g // BB
        bb = g - l * BB
        for half in range(2):
            b = half

            if half == 0:
                issue_gather(gl, 1, 1)
            else:
                @pl.when(gl + 1 < GRP_PER_W)
                def _():
                    issue_gather(gl + 1, 0, 0)

            wait_gather(b)

            @pl.when(gl * 2 + half >= 2)
            def _():
                wait_store(b)

            def tok_body(t, tc):
                ngroups = EMB_DIM // LANES

                def loads(d):
                    return [rows_v[b, h * T + t, pl.ds(d * LANES, LANES)]
                            for h in range(N_HASHES)]

                pipe = [loads(0), loads(1), loads(2)]
                for d in range(ngroups):
                    if d + 3 < ngroups:
                        pipe.append(loads(d + 3))
                    v0, v1, v2, v3 = pipe.pop(0)
                    out_v[b, t, pl.ds(d * LANES, LANES)] = (
                        (v0 + v1) + (v2 + v3))
                return tc

            lax.fori_loop(0, T, tok_body, 0, unroll=2)

            base = (bb * 128 + half * T) * L + l
            for jb in range(T // LANES):
                offs_v[b, pl.ds(jb * LANES, LANES)] = (
                    base + (jb * LANES + lax.iota(jnp.int32, 16)) * L)
            pltpu.async_copy(
                out_v.at[b], out_hbm.at[offs_v.at[b]], ssems[b])
        return carry

    lax.fori_loop(0, GRP_PER_W, grp_body, 0, unroll=False)
    wait_store(0)
    wait_store(1)


def kernel(x, table):
    xz = (x.reshape(BB, 128, L, N_HASHES)
          .transpose(2, 0, 3, 1)
          .reshape(L * BB * N_HASHES, 128))
    out = _bloom_sum(xz, table)
    return out.reshape(B, L, EMB_DIM)


# speedup vs baseline: 15.9187x; 1.0408x over previous
import functools

import jax
import jax.numpy as jnp
from jax import lax
from jax.experimental import pallas as pl
from jax.experimental.pallas import tpu as pltpu
from jax.experimental.pallas import tpu_sc as plsc

N_EMB = 100000
EMB_DIM = 128
N_HASHES = 4
B = 1024
L = 200

N_TOK = B * L
NC, NS, LANES = 2, 16, 16
NW = NC * NS
BB = B // 128
NGRP = L * BB
GRP_PER_W = NGRP // NW
T = 32
Q = 128 // T
NBUF = 4
ZROWS_PER_W = GRP_PER_W * N_HASHES


@functools.partial(
    pl.kernel,
    out_type=jax.ShapeDtypeStruct((N_TOK, EMB_DIM), jnp.float32),
    mesh=plsc.VectorSubcoreMesh(
        core_axis_name="c", subcore_axis_name="s", num_cores=NC,
        num_subcores=NS),
    scratch_types=[
        pltpu.VMEM((ZROWS_PER_W, 128), jnp.int32),
        pltpu.VMEM((NBUF, N_HASHES * T, EMB_DIM), jnp.float32),
        pltpu.VMEM((NBUF, T, EMB_DIM), jnp.float32),
        pltpu.VMEM((NBUF, T), jnp.int32),
        pltpu.SemaphoreType.DMA,
        pltpu.SemaphoreType.DMA,
        pltpu.SemaphoreType.DMA,
        pltpu.SemaphoreType.DMA,
        pltpu.SemaphoreType.DMA,
        pltpu.SemaphoreType.DMA,
        pltpu.SemaphoreType.DMA,
        pltpu.SemaphoreType.DMA,
    ],
)
def _bloom_sum(z_hbm, table_hbm, out_hbm, idx_v, rows_v, out_v, offs_v,
               g0s, g1s, g2s, g3s, s0s, s1s, s2s, s3s):
    wid = lax.axis_index("s") * NC + lax.axis_index("c")
    pltpu.sync_copy(z_hbm.at[pl.ds(wid * ZROWS_PER_W, ZROWS_PER_W)], idx_v)
    g0 = wid * GRP_PER_W
    gsems = [g0s, g1s, g2s, g3s]
    ssems = [s0s, s1s, s2s, s3s]

    def issue_gather(gl, q, b):
        for h in range(N_HASHES):
            pltpu.async_copy(
                table_hbm.at[idx_v.at[gl * N_HASHES + h, pl.ds(q * T, T)]],
                rows_v.at[b].at[pl.ds(h * T, T)], gsems[b])

    def wait_gather(b):
        for h in range(N_HASHES):
            pltpu.make_async_copy(
                table_hbm.at[idx_v.at[h, pl.ds(0, T)]],
                rows_v.at[b].at[pl.ds(h * T, T)], gsems[b]).wait()

    def wait_store(b):
        pltpu.make_async_copy(
            out_v.at[b], out_hbm.at[offs_v.at[b]], ssems[b]).wait()

    issue_gather(0, 0, 0)
    issue_gather(0, 1, 1)
    issue_gather(0, 2, 2)

    def grp_body(gl, carry):
        g = g0 + gl
        l = g // BB
        bb = g - l * BB
        for q in range(Q):
            b = q
            s = gl * Q + q

            qn = (q + 3) % Q
            gn = gl + (q + 3) // Q

            @pl.when(gn < GRP_PER_W)
            def _():
                issue_gather(gn, qn, qn)

            wait_gather(b)

            @pl.when(s >= NBUF)
            def _():
                wait_store(b)

            def tok_body(t, tc):
                ngroups = EMB_DIM // LANES

                def loads(d):
                    return [rows_v[b, h * T + t, pl.ds(d * LANES, LANES)]
                            for h in range(N_HASHES)]

                pipe = [loads(0), loads(1), loads(2)]
                for d in range(ngroups):
                    if d + 3 < ngroups:
                        pipe.append(loads(d + 3))
                    v0, v1, v2, v3 = pipe.pop(0)
                    out_v[b, t, pl.ds(d * LANES, LANES)] = (
                        (v0 + v1) + (v2 + v3))
                return tc

            lax.fori_loop(0, T, tok_body, 0, unroll=2)

            base = (bb * 128 + q * T) * L + l
            for jb in range(T // LANES):
                offs_v[b, pl.ds(jb * LANES, LANES)] = (
                    base + (jb * LANES + lax.iota(jnp.int32, 16)) * L)
            pltpu.async_copy(
                out_v.at[b], out_hbm.at[offs_v.at[b]], ssems[b])
        return carry

    lax.fori_loop(0, GRP_PER_W, grp_body, 0, unroll=False)
    for b in range(NBUF):
        wait_store(b)


def kernel(x, table):
    xz = (x.reshape(BB, 128, L, N_HASHES)
          .transpose(2, 0, 3, 1)
          .reshape(L * BB * N_HASHES, 128))
    out = _bloom_sum(xz, table)
    return out.reshape(B, L, EMB_DIM)
